# Initial kernel scaffold; baseline (speedup 1.0000x reference)
import jax
import jax.numpy as jnp
from jax.experimental import pallas as pl


def kernel(user_emb, news_emb, entity_emb, relation_emb, interact_vals, news_entities, news_relations, neigh_entities, neigh_relations, interact_rows, interact_cols):
    raise NotImplementedError("write your pallas kernel here")


# speedup vs baseline: 4.9878x; 4.9878x over previous
import functools

import jax
import jax.numpy as jnp
from jax import lax
from jax.experimental import pallas as pl
from jax.experimental.pallas import tpu as pltpu
from jax.experimental.pallas import tpu_sc as plsc

NC = 2
NS = 16
NW = NC * NS
GC = 128


def _sc_gather_rows(table, idx, tail_rows):
    B = idx.shape[0]
    D = table.shape[1]
    n_full = (B - tail_rows) // GC
    n_tot = n_full + (1 if tail_rows else 0)
    it = -(-n_tot // NW)
    mesh = plsc.VectorSubcoreMesh(core_axis_name="c", subcore_axis_name="s")

    @functools.partial(
        pl.kernel,
        out_type=jax.ShapeDtypeStruct((B, D), jnp.float32),
        mesh=mesh,
        scratch_types=[
            pltpu.VMEM((2, GC), jnp.int32),
            pltpu.VMEM((2, GC, D), jnp.float32),
            pltpu.SemaphoreType.DMA,
            pltpu.SemaphoreType.DMA,
            pltpu.SemaphoreType.DMA,
            pltpu.SemaphoreType.DMA,
        ],
    )
    def k(table_hbm, idx_hbm, out_hbm, idx_v, rows_v,
          sem_g0, sem_g1, sem_o0, sem_o1):
        wid = lax.axis_index("s") * NC + lax.axis_index("c")
        sem_g = (sem_g0, sem_g1)
        sem_o = (sem_o0, sem_o1)

        def cid_of(j):
            return j * NW + wid

        def start(j, b):
            base = cid_of(j) * GC
            pltpu.sync_copy(idx_hbm.at[pl.ds(base, GC)], idx_v.at[b])
            pltpu.async_copy(table_hbm.at[idx_v.at[b]], rows_v.at[b], sem_g[b])

        @pl.when(cid_of(0) < n_full)
        def _():
            start(0, 0)

        def pair(i, carry):
            for b in range(2):
                j2i = i * 2 + b
                nb = 1 - b

                @pl.when(cid_of(j2i + 1) < n_full)
                def _():
                    @pl.when(j2i >= 1)
                    def _():
                        pltpu.make_async_copy(
                            rows_v.at[nb],
                            out_hbm.at[pl.ds(cid_of(j2i - 1) * GC, GC)],
                            sem_o[nb]).wait()
                    start(j2i + 1, nb)

                @pl.when(cid_of(j2i) < n_full)
                def _():
                    pltpu.make_async_copy(table_hbm.at[idx_v.at[b]],
                                          rows_v.at[b], sem_g[b]).wait()
                    pltpu.async_copy(rows_v.at[b],
                                     out_hbm.at[pl.ds(cid_of(j2i) * GC, GC)],
                                     sem_o[b])
            return carry

        lax.fori_loop(0, (it + 1) // 2, pair, 0)

        vj = lax.max(0, (n_full - wid + NW - 1) // NW)

        @pl.when(vj >= 1)
        def _():
            pltpu.make_async_copy(rows_v.at[0], out_hbm.at[pl.ds(0, GC)],
                                  sem_o0).wait()

        @pl.when(vj >= 2)
        def _():
            pltpu.make_async_copy(rows_v.at[1], out_hbm.at[pl.ds(0, GC)],
                                  sem_o1).wait()

        if tail_rows:
            @pl.when(wid == (n_full % NW))
            def _tail():
                base = n_full * GC
                pltpu.sync_copy(idx_hbm.at[pl.ds(base, tail_rows)],
                                idx_v.at[0].at[pl.ds(0, tail_rows)])
                pltpu.async_copy(
                    table_hbm.at[idx_v.at[0].at[pl.ds(0, tail_rows)]],
                    rows_v.at[0].at[pl.ds(0, tail_rows)], sem_g0).wait()
                pltpu.sync_copy(rows_v.at[0].at[pl.ds(0, tail_rows)],
                                out_hbm.at[pl.ds(base, tail_rows)])

    return k(table, idx)


def _sc_user_partials(news_agg, cols, rows, vals, n_users):
    nnzp = cols.shape[0]
    D = news_agg.shape[1]
    n_chunks = nnzp // (NW * GC)
    zc = 80
    n_zc = n_users // zc
    zit = -(-n_zc // NS)
    mesh = plsc.VectorSubcoreMesh(core_axis_name="c", subcore_axis_name="s")

    @functools.partial(
        pl.kernel,
        out_type=jax.ShapeDtypeStruct((NC * n_users, D), jnp.float32),
        mesh=mesh,
        scratch_types=[
            pltpu.VMEM((2, GC), jnp.int32),
            pltpu.VMEM((2, GC), jnp.int32),
            pltpu.VMEM((2, GC), jnp.float32),
            pltpu.VMEM((2, GC, D), jnp.float32),
            pltpu.VMEM_SHARED((n_users, D), jnp.float32),
            pltpu.SemaphoreType.DMA,
            pltpu.SemaphoreType.DMA,
            pltpu.SemaphoreType.DMA,
            pltpu.SemaphoreType.DMA,
        ],
    )
    def k(news_hbm, cols_hbm, rows_hbm, vals_hbm, out_hbm,
          cols_v, rows_v, vals_v, g_v, accum,
          sem_g0, sem_g1, sem_s0, sem_s1):
        c = lax.axis_index("c")
        s = lax.axis_index("s")
        wid = s * NC + c

        sem_g = (sem_g0, sem_g1)
        sem_s = (sem_s0, sem_s1)
        zero16 = jnp.zeros((16,), jnp.float32)

        def zbody(i, carry):
            for c16 in range(D // 16):
                g_v[0, i, pl.ds(c16 * 16, 16)] = zero16
            return carry

        lax.fori_loop(0, GC, zbody, 0)

        def zcopy(j, carry):
            cid = j * NS + s

            @pl.when(cid < n_zc)
            def _():
                pltpu.sync_copy(g_v.at[0].at[pl.ds(0, zc)],
                                accum.at[pl.ds(cid * zc, zc)])
            return carry

        lax.fori_loop(0, zit, zcopy, 0)
        plsc.subcore_barrier()

        def load_idx(j, b):
            base = (wid * n_chunks + j) * GC
            pltpu.sync_copy(cols_hbm.at[pl.ds(base, GC)], cols_v.at[b])
            pltpu.sync_copy(rows_hbm.at[pl.ds(base, GC)], rows_v.at[b])
            pltpu.sync_copy(vals_hbm.at[pl.ds(base, GC)], vals_v.at[b])

        def scale(b):
            def mbody(i16, carry2):
                val16 = vals_v[b, pl.ds(i16 * 16, 16)]
                for r in range(16):
                    v = val16[r]
                    row = i16 * 16 + r
                    for c16 in range(D // 16):
                        g_v[b, row, pl.ds(c16 * 16, 16)] = (
                            g_v[b, row, pl.ds(c16 * 16, 16)] * v)
                return carry2

            lax.fori_loop(0, GC // 16, mbody, 0)

        load_idx(0, 0)
        pltpu.async_copy(news_hbm.at[cols_v.at[0]], g_v.at[0], sem_g0)

        def pair(i, carry):
            for b in range(2):
                j2 = i * 2 + b
                nb = 1 - b

                @pl.when(j2 + 1 < n_chunks)
                def _():
                    @pl.when(j2 >= 1)
                    def _():
                        pltpu.make_async_copy(
                            g_v.at[nb], accum.at[rows_v.at[nb]], sem_s[nb]).wait()
                    load_idx(j2 + 1, nb)
                    pltpu.async_copy(news_hbm.at[cols_v.at[nb]], g_v.at[nb],
                                     sem_g[nb])

                pltpu.make_async_copy(news_hbm.at[cols_v.at[b]], g_v.at[b],
                                      sem_g[b]).wait()
                scale(b)
                pltpu.async_copy(g_v.at[b], accum.at[rows_v.at[b]], sem_s[b],
                                 add=True)
            return carry

        lax.fori_loop(0, n_chunks // 2, pair, 0)
        pltpu.make_async_copy(g_v.at[0], accum.at[rows_v.at[0]], sem_s0).wait()
        pltpu.make_async_copy(g_v.at[1], accum.at[rows_v.at[1]], sem_s1).wait()
        plsc.subcore_barrier()

        def dump(j, carry):
            cid = j * NS + s

            @pl.when(cid < n_zc)
            def _():
                r0 = cid * zc
                pltpu.sync_copy(accum.at[pl.ds(r0, zc)],
                                out_hbm.at[pl.ds(c * n_users + r0, zc)])
            return carry

        lax.fori_loop(0, zit, dump, 0)

    return k(news_agg, cols, rows, vals)


def _tc_attention_agg(head, tails_t, relidx_t, rel_table, br):
    N, D = head.shape
    K = tails_t.shape[0]
    NREL = rel_table.shape[0]

    def body(h_ref, t_ref, r_ref, rt_ref, o_ref):
        h = h_ref[...]
        rt = rt_ref[...]
        rels = []
        srel = jnp.zeros((br, D), jnp.float32)
        for kk in range(K):
            rk = r_ref[kk]
            ohk = (rk == lax.broadcasted_iota(jnp.int32, (br, NREL), 1)
                   ).astype(jnp.float32)
            relk = jnp.dot(ohk, rt, preferred_element_type=jnp.float32,
                           precision=lax.Precision.HIGHEST)
            rels.append(relk)
            hr = h * relk
            srel = srel + hr * hr
        hn = jnp.sqrt(srel)
        hn16 = hn.astype(jnp.bfloat16).astype(jnp.float32)
        atts = []
        for kk in range(K):
            trk = (t_ref[kk] * rels[kk]).astype(jnp.bfloat16).astype(jnp.float32)
            a = jnp.sum(trk * hn16, axis=1, keepdims=True)
            atts.append(a * a)
        m = atts[0]
        for kk in range(1, K):
            m = jnp.maximum(m, atts[kk])
        l = jnp.zeros((br, 1), jnp.float32)
        acc = jnp.zeros((br, D), jnp.float32)
        for kk in range(K):
            p = jnp.exp(atts[kk] - m)
            l = l + p
            acc = acc + p * t_ref[kk]
        o_ref[...] = acc / l

    return pl.pallas_call(
        body,
        grid=(N // br,),
        in_specs=[
            pl.BlockSpec((br, D), lambda i: (i, 0)),
            pl.BlockSpec((K, br, D), lambda i: (0, i, 0)),
            pl.BlockSpec((K, br, 1), lambda i: (0, i, 0)),
            pl.BlockSpec((NREL, D), lambda i: (0, 0)),
        ],
        out_specs=pl.BlockSpec((br, D), lambda i: (i, 0)),
        out_shape=jax.ShapeDtypeStruct((N, D), jnp.float32),
    )(head, tails_t, relidx_t, rel_table)


def _tc_flash_combine(user_emb, news_agg, partials, bu):
    NU, D = user_emb.shape
    NN = news_agg.shape[0]

    def body(q_ref, k_ref, p_ref, o_ref):
        q = q_ref[...]
        kk = k_ref[...]
        s = lax.dot_general(q, kk, (((1,), (1,)), ((), ())),
                            preferred_element_type=jnp.float32)
        m = jnp.max(s, axis=1, keepdims=True)
        e = jnp.exp(s - m)
        l = jnp.sum(e, axis=1, keepdims=True)
        o = lax.dot_general(e, kk, (((1,), (0,)), ((), ())),
                            preferred_element_type=jnp.float32)
        ua = p_ref[0] + p_ref[1]
        o_ref[...] = ua + (o / l) * ua

    return pl.pallas_call(
        body,
        grid=(NU // bu,),
        in_specs=[
            pl.BlockSpec((bu, D), lambda i: (i, 0)),
            pl.BlockSpec((NN, D), lambda i: (0, 0)),
            pl.BlockSpec((NC, bu, D), lambda i: (0, i, 0)),
        ],
        out_specs=pl.BlockSpec((bu, D), lambda i: (i, 0)),
        out_shape=jax.ShapeDtypeStruct((NU, D), jnp.float32),
    )(user_emb, news_agg, partials)


def kernel(user_emb, news_emb, entity_emb, relation_emb, interact_vals,
           news_entities, news_relations, neigh_entities, neigh_relations,
           interact_rows, interact_cols):
    NU, D = user_emb.shape
    NN = news_emb.shape[0]
    NE = entity_emb.shape[0]
    K = news_entities.shape[1]

    idx_news = news_entities.astype(jnp.int32).T.reshape(-1)
    idx_ent = neigh_entities.astype(jnp.int32).T.reshape(-1)
    tail_news = (NN * K) % GC
    tail_ent = (NE * K) % GC
    t_news = _sc_gather_rows(entity_emb, idx_news, tail_news).reshape(K, NN, D)
    t_ent = _sc_gather_rows(entity_emb, idx_ent, tail_ent).reshape(K, NE, D)

    ri_news = news_relations.astype(jnp.int32).T[:, :, None]
    ri_ent = neigh_relations.astype(jnp.int32).T[:, :, None]

    news_agg = _tc_attention_agg(news_emb, t_news, ri_news, relation_emb, br=400)
    entity_agg = _tc_attention_agg(entity_emb, t_ent, ri_ent, relation_emb, br=400)

    nnz = interact_vals.shape[0]
    chunk = NW * GC * 2
    nnzp = -(-nnz // chunk) * chunk
    cols_p = jnp.zeros((nnzp,), jnp.int32).at[:nnz].set(interact_cols.astype(jnp.int32))
    rows_p = jnp.zeros((nnzp,), jnp.int32).at[:nnz].set(interact_rows.astype(jnp.int32))
    vals_p = jnp.zeros((nnzp,), jnp.float32).at[:nnz].set(interact_vals)
    partials = _sc_user_partials(news_agg, cols_p, rows_p, vals_p, NU)
    partials = partials.reshape(NC, NU, D)

    user_agg = _tc_flash_combine(user_emb, news_agg, partials, bu=400)

    return (news_agg, entity_agg, user_agg)
